# Initial kernel scaffold; baseline (speedup 1.0000x reference)
import jax
import jax.numpy as jnp
from jax.experimental import pallas as pl


def kernel(q, k, w, mask, top_k):
    raise NotImplementedError("write your pallas kernel here")


# speedup vs baseline: 1.0192x; 1.0192x over previous
import functools

import jax
import jax.numpy as jnp
from jax import lax
from jax.experimental import pallas as pl
from jax.experimental.pallas import tpu as pltpu

TOPK = 512
TB = 256


def _score_body(q_ref, k_ref, w_ref, m_ref, out_ref, *, scale):
    tb, h, d = q_ref.shape
    s = k_ref.shape[0]
    kk = k_ref[...]
    acc = None
    for hh in range(h):
        qh = q_ref[:, hh, :] * scale
        logits = lax.dot_general(
            qh, kk, (((1,), (1,)), ((), ())),
            preferred_element_type=jnp.float32)
        term = jnp.maximum(logits, 0.0) * w_ref[:, hh:hh + 1]
        acc = term if acc is None else acc + term
    out_ref[...] = acc + m_ref[...]


def kernel(q, k, w, mask, top_k):
    b, t, h, d = q.shape
    s = k.shape[1]
    scale = d ** -0.5
    q2, k2, w2, m2 = q[0], k[0], w[0], mask[0]

    score = pl.pallas_call(
        functools.partial(_score_body, scale=scale),
        grid=(t // TB,),
        in_specs=[
            pl.BlockSpec((TB, h, d), lambda i: (i, 0, 0)),
            pl.BlockSpec((s, d), lambda i: (0, 0)),
            pl.BlockSpec((TB, h), lambda i: (i, 0)),
            pl.BlockSpec((TB, s), lambda i: (i, 0)),
        ],
        out_specs=pl.BlockSpec((TB, s), lambda i: (i, 0)),
        out_shape=jax.ShapeDtypeStruct((t, s), jnp.float32),
    )(q2, k2, w2, m2)

    score = score[None]
    _, idx = jax.lax.top_k(score, TOPK)
    index_mask = jnp.take_along_axis(mask, idx, axis=-1)
    return score, idx, index_mask
